# baseline (speedup 1.0000x reference)
import functools

import jax
import jax.numpy as jnp
from jax import lax
from jax.experimental import pallas as pl
from jax.experimental.pallas import tpu as pltpu
from jax.experimental.pallas import tpu_sc as plsc

NC = 2
NS = 16
NW = NC * NS
LANES = 16
EB = 128


def _matmul_body(x_ref, th_ref, y_ref):
    y_ref[...] = jnp.dot(x_ref[...], th_ref[...],
                         preferred_element_type=jnp.float32)


def _add_body(a_ref, b_ref, o_ref):
    o_ref[...] = a_ref[...] + b_ref[...]


def _make_scatter_kernel(n_rows, d_out, nbatch):
    rows_per_tile = n_rows // NS
    zr = 128
    nz = rows_per_tile // zr

    mesh = plsc.VectorSubcoreMesh(core_axis_name="c", subcore_axis_name="s",
                                  num_cores=NC, num_subcores=NS)

    @functools.partial(
        pl.kernel,
        out_type=jax.ShapeDtypeStruct((NC, n_rows, d_out), jnp.float32),
        mesh=mesh,
        scratch_types=[
            pltpu.VMEM((nbatch, EB), jnp.int32),
            pltpu.VMEM((nbatch, EB), jnp.int32),
            pltpu.VMEM((nbatch, EB), jnp.float32),
            pltpu.VMEM((EB, d_out), jnp.float32),
            pltpu.VMEM((EB, d_out), jnp.float32),
            pltpu.VMEM((zr, d_out), jnp.float32),
            pltpu.VMEM_SHARED((n_rows, d_out), jnp.float32),
            pltpu.SemaphoreType.DMA,
            pltpu.SemaphoreType.DMA,
            pltpu.SemaphoreType.DMA,
            pltpu.SemaphoreType.DMA,
        ],
        compiler_params=pltpu.CompilerParams(use_tc_tiling_on_sc=False),
    )
    def scatter_kernel(src_hbm, dst_hbm, vals_hbm, y_hbm, out_hbm,
                       src_v, dst_v, vals_v, rows_v0, rows_v1, zero_v,
                       accum, sem0, sem1, ssem0, ssem1):
        c = lax.axis_index("c")
        s = lax.axis_index("s")
        w = s * NC + c

        def zero_body(i, carry):
            for f in range(d_out // LANES):
                zero_v[i, pl.ds(f * LANES, LANES)] = jnp.zeros(
                    (LANES,), jnp.float32)
            return carry
        lax.fori_loop(0, zr, zero_body, 0)
        for k in range(nz):
            pltpu.sync_copy(
                zero_v,
                accum.at[pl.ds(s * rows_per_tile + k * zr, zr), :])

        pltpu.sync_copy(src_hbm.at[w], src_v)
        pltpu.sync_copy(dst_hbm.at[w], dst_v)
        pltpu.sync_copy(vals_hbm.at[w], vals_v)

        plsc.subcore_barrier()

        def scale(rows_v, j):
            for g in range(EB // LANES):
                vals16 = vals_v[j, pl.ds(g * LANES, LANES)]
                for el in range(LANES):
                    e = g * LANES + el
                    v = vals16[el]
                    for fb in range(d_out // LANES):
                        sl = pl.ds(fb * LANES, LANES)
                        rows_v[e, sl] = rows_v[e, sl] * v

        def gather_sync(j, buf, sem):
            pltpu.async_copy(y_hbm.at[src_v.at[j]], buf, sem).wait()

        def scat_start(j, buf, sem):
            pltpu.async_copy(buf, accum.at[dst_v.at[j]], sem, add=True)

        def scat_wait(j, buf, sem):
            pltpu.make_async_copy(buf, accum.at[dst_v.at[j]], sem).wait()

        npair = nbatch // 2
        gather_sync(0, rows_v0, sem0)
        scale(rows_v0, 0)

        def pair_body(i, carry):
            j = 2 * i
            scat_start(j, rows_v0, ssem0)
            gather_sync(j + 1, rows_v1, sem1)
            scale(rows_v1, j + 1)
            scat_wait(j, rows_v0, ssem0)
            scat_start(j + 1, rows_v1, ssem1)

            @pl.when(i + 1 < npair)
            def _():
                gather_sync(j + 2, rows_v0, sem0)
                scale(rows_v0, j + 2)
            scat_wait(j + 1, rows_v1, ssem1)
            return carry
        lax.fori_loop(0, npair, pair_body, 0)

        plsc.subcore_barrier()

        pltpu.sync_copy(
            accum.at[pl.ds(s * rows_per_tile, rows_per_tile), :],
            out_hbm.at[c, pl.ds(s * rows_per_tile, rows_per_tile), :])

    return scatter_kernel


def kernel(L_indices, L_values, x, theta):
    n, d_in = x.shape
    d_out = theta.shape[1]
    nnz = L_values.shape[0]

    rb = 1000
    y = pl.pallas_call(
        _matmul_body,
        grid=(n // rb,),
        in_specs=[
            pl.BlockSpec((rb, d_in), lambda i: (i, 0)),
            pl.BlockSpec((d_in, d_out), lambda i: (0, 0)),
        ],
        out_specs=pl.BlockSpec((rb, d_out), lambda i: (i, 0)),
        out_shape=jax.ShapeDtypeStruct((n, d_out), jnp.float32),
    )(x, theta)

    per_round = NW * EB
    nbatch = 2 * -(-nnz // (2 * per_round))
    nnz_p = nbatch * per_round
    pad = nnz_p - nnz
    pad_idx = jnp.arange(pad, dtype=jnp.int32) % n
    dst = jnp.concatenate([L_indices[0], pad_idx]).reshape(NW, nbatch, EB)
    src = jnp.concatenate([L_indices[1], pad_idx]).reshape(NW, nbatch, EB)
    vals = jnp.pad(L_values, (0, pad)).reshape(NW, nbatch, EB)

    n_pad = -(-n // (NS * 128)) * (NS * 128)
    partials = _make_scatter_kernel(n_pad, d_out, nbatch)(src, dst, vals, y)

    flat = n_pad * d_out
    cb = flat // 10
    out = pl.pallas_call(
        _add_body,
        grid=(10,),
        in_specs=[
            pl.BlockSpec((cb,), lambda i: (i,)),
            pl.BlockSpec((cb,), lambda i: (i,)),
        ],
        out_specs=pl.BlockSpec((cb,), lambda i: (i,)),
        out_shape=jax.ShapeDtypeStruct((flat,), jnp.float32),
    )(partials[0].reshape(flat), partials[1].reshape(flat))
    return out.reshape(n_pad, d_out)[:n]


# speedup vs baseline: 1.0424x; 1.0424x over previous
import functools

import jax
import jax.numpy as jnp
from jax import lax
from jax.experimental import pallas as pl
from jax.experimental.pallas import tpu as pltpu
from jax.experimental.pallas import tpu_sc as plsc

NC = 2
NS = 16
NW = NC * NS
LANES = 16
EB = 128


def _matmul_body(x_ref, th_ref, y_ref):
    y_ref[...] = jnp.dot(x_ref[...], th_ref[...],
                         preferred_element_type=jnp.float32)


def _add_body(a_ref, b_ref, o_ref):
    o_ref[...] = a_ref[...] + b_ref[...]


def _make_scatter_kernel(n_rows, d_out, nbatch):
    rows_per_tile = n_rows // NS
    zr = 128
    nz = rows_per_tile // zr

    mesh = plsc.VectorSubcoreMesh(core_axis_name="c", subcore_axis_name="s",
                                  num_cores=NC, num_subcores=NS)

    @functools.partial(
        pl.kernel,
        out_type=jax.ShapeDtypeStruct((NC, n_rows, d_out), jnp.float32),
        mesh=mesh,
        scratch_types=[
            pltpu.VMEM((nbatch, EB), jnp.int32),
            pltpu.VMEM((nbatch, EB), jnp.int32),
            pltpu.VMEM((nbatch, EB), jnp.float32),
            pltpu.VMEM((EB, d_out), jnp.float32),
            pltpu.VMEM((EB, d_out), jnp.float32),
            pltpu.VMEM((zr, d_out), jnp.float32),
            pltpu.VMEM_SHARED((n_rows, d_out), jnp.float32),
            pltpu.SemaphoreType.DMA,
            pltpu.SemaphoreType.DMA,
            pltpu.SemaphoreType.DMA,
            pltpu.SemaphoreType.DMA,
        ],
        compiler_params=pltpu.CompilerParams(use_tc_tiling_on_sc=False),
    )
    def scatter_kernel(src_hbm, dst_hbm, vals_hbm, y_hbm, out_hbm,
                       src_v, dst_v, vals_v, rows_v0, rows_v1, zero_v,
                       accum, sem0, sem1, ssem0, ssem1):
        c = lax.axis_index("c")
        s = lax.axis_index("s")
        w = s * NC + c

        def zero_body(i, carry):
            for f in range(d_out // LANES):
                zero_v[i, pl.ds(f * LANES, LANES)] = jnp.zeros(
                    (LANES,), jnp.float32)
            return carry
        lax.fori_loop(0, zr, zero_body, 0)
        for k in range(nz):
            pltpu.sync_copy(
                zero_v,
                accum.at[pl.ds(s * rows_per_tile + k * zr, zr), :])

        pltpu.sync_copy(src_hbm.at[w], src_v)
        pltpu.sync_copy(dst_hbm.at[w], dst_v)
        pltpu.sync_copy(vals_hbm.at[w], vals_v)

        plsc.subcore_barrier()

        def scale(rows_v, j):
            for g in range(EB // LANES):
                vals16 = vals_v[j, pl.ds(g * LANES, LANES)]
                for el in range(LANES):
                    e = g * LANES + el
                    v = vals16[el]
                    for fb in range(d_out // LANES):
                        sl = pl.ds(fb * LANES, LANES)
                        rows_v[e, sl] = rows_v[e, sl] * v

        def gather_sync(j, buf, sem):
            pltpu.async_copy(y_hbm.at[src_v.at[j]], buf, sem).wait()

        def scat_start(j, buf, sem):
            pltpu.async_copy(buf, accum.at[dst_v.at[j]], sem, add=True)

        def scat_wait(j, buf, sem):
            pltpu.make_async_copy(buf, accum.at[dst_v.at[j]], sem).wait()

        npair = nbatch // 2
        gather_sync(0, rows_v0, sem0)
        scale(rows_v0, 0)

        def pair_body(i, carry):
            j = 2 * i
            scat_start(j, rows_v0, ssem0)
            gather_sync(j + 1, rows_v1, sem1)
            scale(rows_v1, j + 1)
            scat_wait(j, rows_v0, ssem0)
            scat_start(j + 1, rows_v1, ssem1)

            @pl.when(i + 1 < npair)
            def _():
                gather_sync(j + 2, rows_v0, sem0)
                scale(rows_v0, j + 2)
            scat_wait(j + 1, rows_v1, ssem1)
            return carry
        lax.fori_loop(0, npair, pair_body, 0)

        plsc.subcore_barrier()

        pltpu.sync_copy(
            accum.at[pl.ds(s * rows_per_tile, rows_per_tile), :],
            out_hbm.at[c, pl.ds(s * rows_per_tile, rows_per_tile), :])

    return scatter_kernel


def kernel(L_indices, L_values, x, theta):
    n, d_in = x.shape
    d_out = theta.shape[1]
    nnz = L_values.shape[0]

    rb = 1000
    y = pl.pallas_call(
        _matmul_body,
        grid=(n // rb,),
        in_specs=[
            pl.BlockSpec((rb, d_in), lambda i: (i, 0)),
            pl.BlockSpec((d_in, d_out), lambda i: (0, 0)),
        ],
        out_specs=pl.BlockSpec((rb, d_out), lambda i: (i, 0)),
        out_shape=jax.ShapeDtypeStruct((n, d_out), jnp.float32),
    )(x, theta)

    per_round = NW * EB
    nbatch = 2 * -(-nnz // (2 * per_round))
    nnz_p = nbatch * per_round
    pad = nnz_p - nnz
    pad_idx = jnp.arange(pad, dtype=jnp.int32) % n
    dst = jnp.concatenate([L_indices[0], pad_idx]).reshape(NW, nbatch, EB)
    src = jnp.concatenate([L_indices[1], pad_idx]).reshape(NW, nbatch, EB)
    vals = jnp.pad(L_values, (0, pad)).reshape(NW, nbatch, EB)

    n_pad = -(-n // (NS * 128)) * (NS * 128)
    partials = _make_scatter_kernel(n_pad, d_out, nbatch)(src, dst, vals, y)

    rb2 = n_pad // 10
    out = pl.pallas_call(
        _add_body,
        grid=(10,),
        in_specs=[
            pl.BlockSpec((rb2, d_out), lambda i: (i, 0)),
            pl.BlockSpec((rb2, d_out), lambda i: (i, 0)),
        ],
        out_specs=pl.BlockSpec((rb2, d_out), lambda i: (i, 0)),
        out_shape=jax.ShapeDtypeStruct((n_pad, d_out), jnp.float32),
    )(partials[0], partials[1])
    return out[:n]
